# baseline (speedup 1.0000x reference)
import functools

import jax
import jax.numpy as jnp
from jax import lax
from jax.experimental import pallas as pl
from jax.experimental.pallas import tpu as pltpu
from jax.experimental.pallas import tpu_sc as plsc

_N = 10000
_W = 384
_WS = 320
_RPT = 320
_NW = 32
_NP = _NW * _RPT
_WG = _W // 16


def _memb_body(adjb2, adjbs_flat, srcs, dsts, ebnd, memb_flat,
               slab, gbuf0, gbuf1, sbuf, dbuf0, dbuf1, ebuf, sem0, sem1):
    wid = lax.axis_index("s") * 2 + lax.axis_index("c")
    rlo = wid * _RPT

    pltpu.sync_copy(ebnd.at[pl.ds(wid * 16, 16)], ebuf)
    ebv = ebuf[...]
    elo = ebv[0]
    ehi = ebv[1]

    pltpu.sync_copy(adjbs_flat.at[pl.ds(rlo * _WS, _RPT * _WS)],
                    slab.at[pl.ds(0, _RPT * _WS)])

    c0 = lax.div(elo, 16)
    nch = lax.div(ehi + 15, 16) - c0

    def fire(ci, db, gb, sem):
        base = (c0 + ci) * 16
        pltpu.sync_copy(dsts.at[pl.ds(base, 16)], db)
        pltpu.async_copy(adjb2.at[db], gb, sem)

    def or_chunk(ci, db, gb, sem):
        pltpu.make_async_copy(adjb2.at[db], gb, sem).wait()
        base = (c0 + ci) * 16
        pltpu.sync_copy(srcs.at[pl.ds(base, 16)], sbuf)
        sv = sbuf[...]
        for j in range(16):
            s = sv[j]
            ej = base + j
            act = jnp.logical_and(ej >= elo, ej < ehi)
            off = jnp.where(act, s - rlo, _RPT) * _WS

            def orw(w, c):
                o = off + w * 64
                g = w * 64
                for u in range(4):
                    slab[pl.ds(o + u * 16, 16)] = (
                        slab[pl.ds(o + u * 16, 16)] | gb[j, pl.ds(g + u * 16, 16)])
                return c

            lax.fori_loop(0, _WS // 64, orw, 0)

    @pl.when(nch > 0)
    def _():
        fire(0, dbuf0, gbuf0, sem0)

    def pair(ip, carry):
        i0 = ip * 2

        @pl.when(i0 + 1 < nch)
        def _():
            fire(i0 + 1, dbuf1, gbuf1, sem1)

        or_chunk(i0, dbuf0, gbuf0, sem0)

        @pl.when(i0 + 2 < nch)
        def _():
            fire(i0 + 2, dbuf0, gbuf0, sem0)

        @pl.when(i0 + 1 < nch)
        def _():
            or_chunk(i0 + 1, dbuf1, gbuf1, sem1)

        return carry

    lax.fori_loop(0, lax.div(nch + 1, 2), pair, 0)

    pltpu.sync_copy(slab.at[pl.ds(0, _RPT * _WS)],
                    memb_flat.at[pl.ds(rlo * _WS, _RPT * _WS)])


@jax.jit
def _memb_call(adjb, srcs, dsts, ebnd):
    mesh = plsc.VectorSubcoreMesh(core_axis_name="c", subcore_axis_name="s")
    fn = pl.kernel(
        _memb_body,
        out_type=jax.ShapeDtypeStruct((_NP * _WS,), jnp.int32),
        mesh=mesh,
        scratch_types=[
            pltpu.VMEM(((_RPT + 1) * _WS,), jnp.int32),
            pltpu.VMEM((16, _W), jnp.int32),
            pltpu.VMEM((16, _W), jnp.int32),
            pltpu.VMEM((16,), jnp.int32),
            pltpu.VMEM((16,), jnp.int32),
            pltpu.VMEM((16,), jnp.int32),
            pltpu.VMEM((16,), jnp.int32),
            pltpu.SemaphoreType.DMA,
            pltpu.SemaphoreType.DMA,
        ],
    )
    return fn(adjb, adjb[:, :_WS].reshape(-1), srcs, dsts, ebnd)


def _qgather_body(tab, idx, out, gb0, gb1, db0, db1, sem0, sem1):
    wid = lax.axis_index("s") * 2 + lax.axis_index("c")
    nch = idx.shape[0] // 16
    ncpt = nch // 32
    extra = nch % 32
    clo = wid * ncpt + jnp.minimum(wid, extra)
    chi = clo + ncpt + (wid < extra)

    def fire(ci, db, gb, sem):
        pltpu.sync_copy(idx.at[pl.ds(ci * 16, 16)], db)
        pltpu.async_copy(tab.at[db], gb, sem)

    def flush(ci, db, gb, sem):
        pltpu.make_async_copy(tab.at[db], gb, sem).wait()
        pltpu.sync_copy(gb, out.at[pl.ds(ci * 16, 16)])

    @pl.when(chi > clo)
    def _():
        fire(clo, db0, gb0, sem0)

    def pair(ip, carry):
        i0 = clo + ip * 2

        @pl.when(i0 + 1 < chi)
        def _():
            fire(i0 + 1, db1, gb1, sem1)

        flush(i0, db0, gb0, sem0)

        @pl.when(i0 + 2 < chi)
        def _():
            fire(i0 + 2, db0, gb0, sem0)

        @pl.when(i0 + 1 < chi)
        def _():
            flush(i0 + 1, db1, gb1, sem1)

        return carry

    lax.fori_loop(0, lax.div(chi - clo + 1, 2), pair, 0)


@jax.jit
def _qgather_call(tab, idx):
    B = idx.shape[0]
    mesh = plsc.VectorSubcoreMesh(core_axis_name="c", subcore_axis_name="s")
    fn = pl.kernel(
        _qgather_body,
        out_type=jax.ShapeDtypeStruct((B, _W), jnp.int32),
        mesh=mesh,
        scratch_types=[
            pltpu.VMEM((16, _W), jnp.int32),
            pltpu.VMEM((16, _W), jnp.int32),
            pltpu.VMEM((16,), jnp.int32),
            pltpu.VMEM((16,), jnp.int32),
            pltpu.SemaphoreType.DMA,
            pltpu.SemaphoreType.DMA,
        ],
    )
    return fn(tab, idx)


def _rowcnt_body(memb_ref, out_ref):
    pc = lax.population_count(memb_ref[...])
    out_ref[...] = jnp.sum(pc.reshape(8, 128, _WS), axis=2)


@jax.jit
def _rowcnt_call(memb):
    return pl.pallas_call(
        _rowcnt_body,
        grid=(_NP // 1024,),
        in_specs=[pl.BlockSpec((1024, _WS), lambda i: (i, 0))],
        out_specs=pl.BlockSpec((8, 128), lambda i: (i, 0)),
        out_shape=jax.ShapeDtypeStruct((_NP // 128, 128), jnp.int32),
    )(memb).reshape(-1)


def _build_bits(rows, bits):
    key = rows * _N + bits
    first = jnp.concatenate([jnp.array([True]), key[1:] != key[:-1]])
    val = jnp.where(first, jnp.left_shift(jnp.int32(1), bits & 31), 0)
    adj = jnp.zeros((_NP, _W), jnp.int32)
    return adj.at[rows, jnp.right_shift(bits, 5)].add(val)


def _ebounds(rows_sorted):
    lo = jnp.searchsorted(rows_sorted, jnp.arange(_NW, dtype=jnp.int32) * _RPT)
    hi = jnp.searchsorted(rows_sorted,
                          jnp.arange(_NW, dtype=jnp.int32) * _RPT + _RPT)
    eb = jnp.zeros((_NW, 16), jnp.int32)
    eb = eb.at[:, 0].set(lo.astype(jnp.int32)).at[:, 1].set(hi.astype(jnp.int32))
    return eb.reshape(-1)


def kernel(x, ei, pos1, pos2, feat, Wg1, bg1, Wg2, bg2, W1, b1, W2, b2, W3, b3,
           Wd, bd):
    n = _N
    src = ei[0].astype(jnp.int32)
    dst = ei[1].astype(jnp.int32)
    E = src.shape[0]
    pos = pos1[pos2][:, 0].reshape(-1, 2)
    qu = pos[:, 0].astype(jnp.int32)
    qw = pos[:, 1].astype(jnp.int32)
    nq = qu.shape[0]

    key = src * n + dst
    ks = jnp.sort(key)
    srcs = ks // n
    dsts = ks % n
    keyT = dst * n + src
    ksT = jnp.sort(keyT)
    srcsT = ksT // n
    dstsT = ksT % n

    roffT = jnp.searchsorted(srcsT, jnp.arange(n + 1, dtype=jnp.int32))
    deg = (roffT[1:] - roffT[:-1]).astype(jnp.float32) + 1.0
    dinv = lax.rsqrt(deg)

    def gcn_layer(xin, Wm, bv):
        h = xin @ Wm
        agg = jnp.zeros((n, h.shape[1]), h.dtype).at[dsts].add(
            (h * dinv[:, None])[srcs])
        return agg * dinv[:, None] + h * (dinv * dinv)[:, None] + bv

    xn = gcn_layer(feat, Wg1, bg1)
    xn = gcn_layer(xn, Wg2, bg2)
    xx = xn[qu] * xn[qw]

    vals = jnp.concatenate([xn[srcs], xn[dsts]], 1)
    xes = vals @ W1 + b1
    muls = vals @ W2 + b2
    first = jnp.concatenate([jnp.array([True]), ks[1:] != ks[:-1]])
    last = jnp.concatenate([ks[1:] != ks[:-1], jnp.array([True])])
    idx = jnp.arange(E, dtype=jnp.int32)
    gstart = lax.cummax(jnp.where(first, idx, 0))
    gend = E - 1 - lax.cummax(jnp.where(last, E - 1 - idx, 0)[::-1])[::-1]
    gcnt_at = (gend - gstart + 1).astype(jnp.float32)
    psum = jnp.concatenate([jnp.zeros((1, muls.shape[1])), jnp.cumsum(muls, 0)])
    musum_at = psum[gend + 1] - psum[gstart]

    adjb = _build_bits(srcs, dsts)
    adjtb = _build_bits(srcsT, dstsT)
    memb = _memb_call(adjb, srcs, dsts, _ebounds(srcs)).reshape(_NP, _WS)
    membt = _memb_call(adjtb, srcsT, dstsT, _ebounds(srcsT)).reshape(_NP, _WS)
    rowcnt = _rowcnt_call(memb)
    colcnt = _rowcnt_call(membt)
    memb384 = jnp.zeros((_NP, _W), jnp.int32).at[:, :_WS].set(memb)
    membt384 = jnp.zeros((_NP, _W), jnp.int32).at[:, :_WS].set(membt)

    cumrow_excl = jnp.concatenate([jnp.zeros((1,), jnp.int32),
                                   jnp.cumsum(rowcnt[:n])[:-1].astype(jnp.int32)])
    rowq = _qgather_call(memb384, qu)[:, :_WS]
    wq = jnp.right_shift(qw, 5)
    bq = qw & 31
    pcrow = lax.population_count(rowq)
    before = jnp.sum(jnp.where(jnp.arange(_WS)[None, :] < wq[:, None], pcrow, 0), 1)
    word_q = jnp.take_along_axis(rowq, wq[:, None], 1)[:, 0]
    mask_incl = ~jnp.left_shift(jnp.int32(-2), bq)
    inc = lax.population_count(word_q & mask_incl)
    rank = cumrow_excl[qu] + before + inc - 1
    ok = (jnp.right_shift(word_q, bq) & 1).astype(jnp.float32)

    cumcol = jnp.cumsum(colcnt[:n]).astype(jnp.int32)
    nnz = cumcol[-1]
    t = jnp.clip(rank + 1, 1, nnz)
    wj = jnp.searchsorted(cumcol, t, side="left").astype(jnp.int32)
    cumcol_excl = cumcol - colcnt[:n]
    rr = t - cumcol_excl[wj]
    colrow = _qgather_call(membt384, wj)[:, :_WS]
    pccol = lax.population_count(colrow)
    ccum = jnp.cumsum(pccol, 1)
    widx = jnp.sum((ccum < rr[:, None]).astype(jnp.int32), 1)
    prev = jnp.where(widx > 0,
                     jnp.take_along_axis(ccum, jnp.maximum(widx - 1, 0)[:, None],
                                         1)[:, 0], 0)
    need = rr - prev
    wordj = jnp.take_along_axis(colrow, widx[:, None], 1)[:, 0]
    bitsm = jnp.right_shift(wordj[:, None], jnp.arange(32)[None, :]) & 1
    bcum = jnp.cumsum(bitsm, 1)
    bsel = jnp.argmax((bcum == need[:, None]) & (bitsm == 1), 1).astype(jnp.int32)
    uj = widx * 32 + bsel

    pu = jnp.concatenate([qu, uj])
    pw = jnp.concatenate([qw, wj])
    P = 2 * nq

    roff = jnp.searchsorted(srcs, jnp.arange(n + 1, dtype=jnp.int32)).astype(jnp.int32)
    plo = roff[pu]
    phi = roff[pu + 1]
    CH = 32
    maxdeg = jnp.max(roff[1:] - roff[:-1])
    niter = (jnp.max(phi - plo) + CH - 1) // CH
    nsteps = jnp.where(maxdeg > 1, 33 - lax.clz(maxdeg - 1), 1)

    def citer(it, acc):
        e = plo[:, None] + it * CH + jnp.arange(CH)[None, :]
        m = e < phi[:, None]
        e = jnp.where(m, e, 0)
        k = dsts[e]
        target = pw[:, None]
        lo = roff[k]
        hi0 = roff[k + 1]

        def bstep(_, lohi):
            blo, bhi = lohi
            mid = jnp.right_shift(blo + bhi, 1)
            dm = dsts[jnp.minimum(mid, E - 1)]
            go = dm < target
            return jnp.where(go, mid + 1, blo), jnp.where(go, bhi, mid)

        lo, _hi = lax.fori_loop(0, nsteps, bstep, (lo, hi0))
        lof = jnp.minimum(lo, E - 1)
        hit = (lo < hi0) & (dsts[lof] == target) & m
        big = jnp.int32(2147483647)
        code = jnp.where(hit, lof * CH + jnp.arange(CH)[None, :], big)
        codes = jnp.sort(code, axis=1)
        nh = jnp.max(jnp.sum(hit.astype(jnp.int32), 1))

        def hloop(h, a2):
            chv = lax.dynamic_slice_in_dim(codes, h, 1, axis=1)[:, 0]
            valid = chv != big
            chs = jnp.where(valid, chv, 0)
            lo_h = chs // CH
            e_h = plo + it * CH + chs % CH
            return a2 + xes[e_h] * musum_at[lo_h] * valid[:, None]

        return lax.fori_loop(0, nh, hloop, acc)

    C = lax.fori_loop(0, niter, citer, jnp.zeros((P, xes.shape[1]), jnp.float32))

    def edge_cnt(a, b):
        kk = a * n + b
        lo = jnp.minimum(jnp.searchsorted(ks, kk), E - 1)
        return jnp.where(ks[lo] == kk, gcnt_at[lo], 0.0)

    cq = edge_cnt(qu, qw)
    cj = edge_cnt(uj, wj)
    vq = jnp.concatenate([C[:nq], cq[:, None]], 1)
    vj = jnp.concatenate([C[nq:], cj[:, None]], 1)
    x3q = vq @ W3 + b3
    x3j = vj @ W3 + b3
    xq = x3q * x3j * ok[:, None]
    return jnp.concatenate([xq, xx], 1) @ Wd + bd


# speedup vs baseline: 1.0192x; 1.0192x over previous
import jax
import jax.numpy as jnp
from jax import lax
from jax.experimental import pallas as pl
from jax.experimental.pallas import tpu as pltpu
from jax.experimental.pallas import tpu_sc as plsc

_N = 10000
_W = 384
_WS = 320
_RPT = 320
_NW = 32
_NP = _NW * _RPT


def _memb_body(adjb2, adjbs_flat, srcs, dsts, ebnd, memb_flat,
               slab, gbuf0, gbuf1, sbuf, dbuf0, dbuf1, ebuf, sem0, sem1):
    wid = lax.axis_index("s") * 2 + lax.axis_index("c")
    rlo = wid * _RPT

    pltpu.sync_copy(ebnd.at[pl.ds(wid * 16, 16)], ebuf)
    ebv = ebuf[...]
    elo = ebv[0]
    ehi = ebv[1]

    pltpu.sync_copy(adjbs_flat.at[pl.ds(rlo * _WS, _RPT * _WS)],
                    slab.at[pl.ds(0, _RPT * _WS)])

    c0 = lax.div(elo, 16)
    nch = lax.div(ehi + 15, 16) - c0

    def fire(ci, db, gb, sem):
        base = (c0 + ci) * 16
        pltpu.sync_copy(dsts.at[pl.ds(base, 16)], db)
        pltpu.async_copy(adjb2.at[db], gb, sem)

    def or_chunk(ci, db, gb, sem):
        pltpu.make_async_copy(adjb2.at[db], gb, sem).wait()
        base = (c0 + ci) * 16
        pltpu.sync_copy(srcs.at[pl.ds(base, 16)], sbuf)
        sv = sbuf[...]
        for j in range(16):
            s = sv[j]
            ej = base + j
            act = jnp.logical_and(ej >= elo, ej < ehi)
            off = jnp.where(act, s - rlo, _RPT) * _WS

            def orw(w, c):
                o = off + w * 64
                g = w * 64
                for u in range(4):
                    slab[pl.ds(o + u * 16, 16)] = (
                        slab[pl.ds(o + u * 16, 16)] | gb[j, pl.ds(g + u * 16, 16)])
                return c

            lax.fori_loop(0, _WS // 64, orw, 0)

    @pl.when(nch > 0)
    def _():
        fire(0, dbuf0, gbuf0, sem0)

    def pair(ip, carry):
        i0 = ip * 2

        @pl.when(i0 + 1 < nch)
        def _():
            fire(i0 + 1, dbuf1, gbuf1, sem1)

        or_chunk(i0, dbuf0, gbuf0, sem0)

        @pl.when(i0 + 2 < nch)
        def _():
            fire(i0 + 2, dbuf0, gbuf0, sem0)

        @pl.when(i0 + 1 < nch)
        def _():
            or_chunk(i0 + 1, dbuf1, gbuf1, sem1)

        return carry

    lax.fori_loop(0, lax.div(nch + 1, 2), pair, 0)

    pltpu.sync_copy(slab.at[pl.ds(0, _RPT * _WS)],
                    memb_flat.at[pl.ds(rlo * _WS, _RPT * _WS)])


@jax.jit
def _memb_call(adjb, srcs, dsts, ebnd):
    mesh = plsc.VectorSubcoreMesh(core_axis_name="c", subcore_axis_name="s")
    fn = pl.kernel(
        _memb_body,
        out_type=jax.ShapeDtypeStruct((_NP * _WS,), jnp.int32),
        mesh=mesh,
        scratch_types=[
            pltpu.VMEM(((_RPT + 1) * _WS,), jnp.int32),
            pltpu.VMEM((16, _W), jnp.int32),
            pltpu.VMEM((16, _W), jnp.int32),
            pltpu.VMEM((16,), jnp.int32),
            pltpu.VMEM((16,), jnp.int32),
            pltpu.VMEM((16,), jnp.int32),
            pltpu.VMEM((16,), jnp.int32),
            pltpu.SemaphoreType.DMA,
            pltpu.SemaphoreType.DMA,
        ],
    )
    return fn(adjb, adjb[:, :_WS].reshape(-1), srcs, dsts, ebnd)


def _qgather_body(tab, idx, out, gb0, gb1, db0, db1, sem0, sem1):
    wid = lax.axis_index("s") * 2 + lax.axis_index("c")
    nch = idx.shape[0] // 16
    ncpt = nch // 32
    extra = nch % 32
    clo = wid * ncpt + jnp.minimum(wid, extra)
    chi = clo + ncpt + (wid < extra)

    def fire(ci, db, gb, sem):
        pltpu.sync_copy(idx.at[pl.ds(ci * 16, 16)], db)
        pltpu.async_copy(tab.at[db], gb, sem)

    def flush(ci, db, gb, sem):
        pltpu.make_async_copy(tab.at[db], gb, sem).wait()
        pltpu.sync_copy(gb, out.at[pl.ds(ci * 16, 16)])

    @pl.when(chi > clo)
    def _():
        fire(clo, db0, gb0, sem0)

    def pair(ip, carry):
        i0 = clo + ip * 2

        @pl.when(i0 + 1 < chi)
        def _():
            fire(i0 + 1, db1, gb1, sem1)

        flush(i0, db0, gb0, sem0)

        @pl.when(i0 + 2 < chi)
        def _():
            fire(i0 + 2, db0, gb0, sem0)

        @pl.when(i0 + 1 < chi)
        def _():
            flush(i0 + 1, db1, gb1, sem1)

        return carry

    lax.fori_loop(0, lax.div(chi - clo + 1, 2), pair, 0)


@jax.jit
def _qgather_call(tab, idx):
    B = idx.shape[0]
    mesh = plsc.VectorSubcoreMesh(core_axis_name="c", subcore_axis_name="s")
    fn = pl.kernel(
        _qgather_body,
        out_type=jax.ShapeDtypeStruct((B, _W), jnp.int32),
        mesh=mesh,
        scratch_types=[
            pltpu.VMEM((16, _W), jnp.int32),
            pltpu.VMEM((16, _W), jnp.int32),
            pltpu.VMEM((16,), jnp.int32),
            pltpu.VMEM((16,), jnp.int32),
            pltpu.SemaphoreType.DMA,
            pltpu.SemaphoreType.DMA,
        ],
    )
    return fn(tab, idx)


def _rowcnt_body(memb_ref, out_ref):
    pc = lax.population_count(memb_ref[...])
    out_ref[...] = jnp.sum(pc.reshape(8, 128, _WS), axis=2)


@jax.jit
def _rowcnt_call(memb):
    return pl.pallas_call(
        _rowcnt_body,
        grid=(_NP // 1024,),
        in_specs=[pl.BlockSpec((1024, _WS), lambda i: (i, 0))],
        out_specs=pl.BlockSpec((8, 128), lambda i: (i, 0)),
        out_shape=jax.ShapeDtypeStruct((_NP // 128, 128), jnp.int32),
    )(memb).reshape(-1)


def _build_bits(rows, bits):
    key = rows * _N + bits
    first = jnp.concatenate([jnp.array([True]), key[1:] != key[:-1]])
    val = jnp.where(first, jnp.left_shift(jnp.int32(1), bits & 31), 0)
    adj = jnp.zeros((_NP, _W), jnp.int32)
    return adj.at[rows, jnp.right_shift(bits, 5)].add(val)


def _ebounds(rows_sorted):
    lo = jnp.searchsorted(rows_sorted, jnp.arange(_NW, dtype=jnp.int32) * _RPT)
    hi = jnp.searchsorted(rows_sorted,
                          jnp.arange(_NW, dtype=jnp.int32) * _RPT + _RPT)
    eb = jnp.zeros((_NW, 16), jnp.int32)
    eb = eb.at[:, 0].set(lo.astype(jnp.int32)).at[:, 1].set(hi.astype(jnp.int32))
    return eb.reshape(-1)


def kernel(x, ei, pos1, pos2, feat, Wg1, bg1, Wg2, bg2, W1, b1, W2, b2, W3, b3,
           Wd, bd):
    n = _N
    src = ei[0].astype(jnp.int32)
    dst = ei[1].astype(jnp.int32)
    E = src.shape[0]
    pos = pos1[pos2][:, 0].reshape(-1, 2)
    qu = pos[:, 0].astype(jnp.int32)
    qw = pos[:, 1].astype(jnp.int32)
    nq = qu.shape[0]

    key = src * n + dst
    ks = jnp.sort(key)
    srcs = ks // n
    dsts = ks % n
    keyT = dst * n + src
    ksT = jnp.sort(keyT)
    srcsT = ksT // n
    dstsT = ksT % n

    roffT = jnp.searchsorted(srcsT, jnp.arange(n + 1, dtype=jnp.int32))
    deg = (roffT[1:] - roffT[:-1]).astype(jnp.float32) + 1.0
    dinv = lax.rsqrt(deg)

    def gcn_layer(xin, Wm, bv):
        h = xin @ Wm
        agg = jnp.zeros((n, h.shape[1]), h.dtype).at[dsts].add(
            (h * dinv[:, None])[srcs])
        return agg * dinv[:, None] + h * (dinv * dinv)[:, None] + bv

    xn = gcn_layer(feat, Wg1, bg1)
    xn = gcn_layer(xn, Wg2, bg2)
    xx = xn[qu] * xn[qw]

    vals = jnp.concatenate([xn[srcs], xn[dsts]], 1)
    xes = vals @ W1 + b1
    muls = vals @ W2 + b2
    first = jnp.concatenate([jnp.array([True]), ks[1:] != ks[:-1]])
    gid = jnp.cumsum(first.astype(jnp.int32)) - 1
    gsum = jax.ops.segment_sum(muls, gid, num_segments=E)
    musum_at = gsum[gid]
    gcnt = jax.ops.segment_sum(jnp.ones((E,), jnp.float32), gid, num_segments=E)
    gcnt_at = gcnt[gid]

    adjb = _build_bits(srcs, dsts)
    adjtb = _build_bits(srcsT, dstsT)
    memb = _memb_call(adjb, srcs, dsts, _ebounds(srcs)).reshape(_NP, _WS)
    membt = _memb_call(adjtb, srcsT, dstsT, _ebounds(srcsT)).reshape(_NP, _WS)
    rowcnt = _rowcnt_call(memb)
    colcnt = _rowcnt_call(membt)
    memb384 = jnp.zeros((_NP, _W), jnp.int32).at[:, :_WS].set(memb)
    membt384 = jnp.zeros((_NP, _W), jnp.int32).at[:, :_WS].set(membt)

    cumrow_excl = jnp.concatenate([jnp.zeros((1,), jnp.int32),
                                   jnp.cumsum(rowcnt[:n])[:-1].astype(jnp.int32)])
    rowq = _qgather_call(memb384, qu)[:, :_WS]
    wq = jnp.right_shift(qw, 5)
    bq = qw & 31
    pcrow = lax.population_count(rowq)
    before = jnp.sum(jnp.where(jnp.arange(_WS)[None, :] < wq[:, None], pcrow, 0), 1)
    word_q = jnp.take_along_axis(rowq, wq[:, None], 1)[:, 0]
    mask_incl = ~jnp.left_shift(jnp.int32(-2), bq)
    inc = lax.population_count(word_q & mask_incl)
    rank = cumrow_excl[qu] + before + inc - 1
    ok = (jnp.right_shift(word_q, bq) & 1).astype(jnp.float32)

    cumcol = jnp.cumsum(colcnt[:n]).astype(jnp.int32)
    nnz = cumcol[-1]
    t = jnp.clip(rank + 1, 1, nnz)
    wj = jnp.searchsorted(cumcol, t, side="left").astype(jnp.int32)
    cumcol_excl = cumcol - colcnt[:n]
    rr = t - cumcol_excl[wj]
    colrow = _qgather_call(membt384, wj)[:, :_WS]
    pccol = lax.population_count(colrow)
    ccum = jnp.cumsum(pccol, 1)
    widx = jnp.sum((ccum < rr[:, None]).astype(jnp.int32), 1)
    prev = jnp.where(widx > 0,
                     jnp.take_along_axis(ccum, jnp.maximum(widx - 1, 0)[:, None],
                                         1)[:, 0], 0)
    need = rr - prev
    wordj = jnp.take_along_axis(colrow, widx[:, None], 1)[:, 0]
    bitsm = jnp.right_shift(wordj[:, None], jnp.arange(32)[None, :]) & 1
    bcum = jnp.cumsum(bitsm, 1)
    bsel = jnp.argmax((bcum == need[:, None]) & (bitsm == 1), 1).astype(jnp.int32)
    uj = widx * 32 + bsel

    pu = jnp.concatenate([qu, uj])
    pw = jnp.concatenate([qw, wj])
    P = 2 * nq

    roff = jnp.searchsorted(srcs, jnp.arange(n + 1, dtype=jnp.int32)).astype(jnp.int32)
    plo = roff[pu]
    phi = roff[pu + 1]
    CH = 32
    maxdeg = jnp.max(roff[1:] - roff[:-1])
    niter = (jnp.max(phi - plo) + CH - 1) // CH
    nsteps = jnp.where(maxdeg > 1, 33 - lax.clz(maxdeg - 1), 1)

    def citer(it, acc):
        e = plo[:, None] + it * CH + jnp.arange(CH)[None, :]
        m = e < phi[:, None]
        e = jnp.where(m, e, 0)
        k = dsts[e]
        target = pw[:, None]
        lo = roff[k]
        hi0 = roff[k + 1]

        def bstep(_, lohi):
            blo, bhi = lohi
            mid = jnp.right_shift(blo + bhi, 1)
            dm = dsts[jnp.minimum(mid, E - 1)]
            go = dm < target
            return jnp.where(go, mid + 1, blo), jnp.where(go, bhi, mid)

        lo, _hi = lax.fori_loop(0, nsteps, bstep, (lo, hi0))
        lof = jnp.minimum(lo, E - 1)
        hit = (lo < hi0) & (dsts[lof] == target) & m
        big = jnp.int32(2147483647)
        code = jnp.where(hit, lof * CH + jnp.arange(CH)[None, :], big)
        codes = jnp.sort(code, axis=1)
        nh = jnp.max(jnp.sum(hit.astype(jnp.int32), 1))

        def hloop(h, a2):
            chv = lax.dynamic_slice_in_dim(codes, h, 1, axis=1)[:, 0]
            valid = chv != big
            chs = jnp.where(valid, chv, 0)
            lo_h = chs // CH
            e_h = plo + it * CH + chs % CH
            return a2 + xes[e_h] * musum_at[lo_h] * valid[:, None]

        return lax.fori_loop(0, nh, hloop, acc)

    C = lax.fori_loop(0, niter, citer, jnp.zeros((P, xes.shape[1]), jnp.float32))

    def edge_cnt(a, b):
        kk = a * n + b
        lo = jnp.minimum(jnp.searchsorted(ks, kk), E - 1)
        return jnp.where(ks[lo] == kk, gcnt_at[lo], 0.0)

    cq = edge_cnt(qu, qw)
    cj = edge_cnt(uj, wj)
    vq = jnp.concatenate([C[:nq], cq[:, None]], 1)
    vj = jnp.concatenate([C[nq:], cj[:, None]], 1)
    x3q = vq @ W3 + b3
    x3j = vj @ W3 + b3
    xq = x3q * x3j * ok[:, None]
    return jnp.concatenate([xq, xx], 1) @ Wd + bd
